# Initial kernel scaffold; baseline (speedup 1.0000x reference)
import jax
import jax.numpy as jnp
from jax.experimental import pallas as pl


def kernel(x, token_table, pos_table):
    raise NotImplementedError("write your pallas kernel here")


# speedup vs baseline: 1.4157x; 1.4157x over previous
import functools

import jax
import jax.numpy as jnp
from jax import lax
from jax.experimental import pallas as pl
from jax.experimental.pallas import tpu as pltpu
from jax.experimental.pallas import tpu_sc as plsc

MAXLEN = 200
EMB = 32
NUM_CORES = 2
NUM_SUBCORES = 16
NW = NUM_CORES * NUM_SUBCORES

CHUNK_BROWS = 4
CHUNK = CHUNK_BROWS * MAXLEN


def _emb_body(n_chunks, table, idxf, pos, out, idx_v, rows_v, pos_v, sem):
    cid = lax.axis_index("c")
    sid = lax.axis_index("s")
    wid = sid * NUM_CORES + cid
    b_per_w = n_chunks * CHUNK
    base = wid * b_per_w

    pltpu.sync_copy(pos, pos_v)
    pltpu.sync_copy(idxf.at[pl.ds(base, b_per_w)], idx_v)

    def chunk_body(g, carry):
        off = g * CHUNK
        pltpu.async_copy(
            table.at[idx_v.at[pl.ds(off, CHUNK)]], rows_v, sem
        ).wait()

        def pos_body(l, c):
            p0 = pos_v[l, pl.ds(0, 16)]
            p1 = pos_v[l, pl.ds(16, 16)]
            for r in range(CHUNK_BROWS):
                row = r * MAXLEN + l
                rows_v[row, pl.ds(0, 16)] = rows_v[row, pl.ds(0, 16)] + p0
                rows_v[row, pl.ds(16, 16)] = rows_v[row, pl.ds(16, 16)] + p1
            return c

        lax.fori_loop(0, MAXLEN, pos_body, 0)
        pltpu.sync_copy(rows_v, out.at[pl.ds(base + off, CHUNK)])
        return carry

    lax.fori_loop(0, n_chunks, chunk_body, 0)


@functools.partial(jax.jit, static_argnames=())
def kernel(x, token_table, pos_table):
    batch, maxlen = x.shape
    emb = token_table.shape[1]
    nflat = batch * maxlen
    b_per_w = nflat // NW
    n_chunks = b_per_w // CHUNK

    idx_flat = x.reshape(nflat).astype(jnp.int32)

    mesh = plsc.VectorSubcoreMesh(
        core_axis_name="c", subcore_axis_name="s",
        num_cores=NUM_CORES, num_subcores=NUM_SUBCORES,
    )
    out_flat = pl.kernel(
        functools.partial(_emb_body, n_chunks),
        out_type=jax.ShapeDtypeStruct((nflat, emb), jnp.float32),
        mesh=mesh,
        scratch_types=[
            pltpu.VMEM((b_per_w,), jnp.int32),
            pltpu.VMEM((CHUNK, emb), jnp.float32),
            pltpu.VMEM((MAXLEN, emb), jnp.float32),
            pltpu.SemaphoreType.DMA,
        ],
        compiler_params=pltpu.CompilerParams(use_tc_tiling_on_sc=False),
        name="token_pos_embed_sc",
    )(token_table, idx_flat, pos_table)

    return out_flat.reshape(batch, maxlen, emb)
